# Initial kernel scaffold; baseline (speedup 1.0000x reference)
import jax
import jax.numpy as jnp
from jax.experimental import pallas as pl


def kernel(input_ids, token_table, position_table):
    raise NotImplementedError("write your pallas kernel here")


# speedup vs baseline: 2.8490x; 2.8490x over previous
import functools

import jax
import jax.numpy as jnp
from jax import lax
from jax.experimental import pallas as pl
from jax.experimental.pallas import tpu as pltpu
from jax.experimental.pallas import tpu_sc as plsc


def _build(V, D, N, S, NC, NS):
  NW = NC * NS
  NR = N // NW
  CH = 2 * S
  NCH = NR // CH
  NBUF = 4
  K = 80
  NG = CH // K
  REP = CH // S
  NL = 16
  assert N % NW == 0 and NR % CH == 0 and NCH % NBUF == 0 and CH % K == 0
  assert K % 8 == 0 and D % NL == 0

  mesh = plsc.VectorSubcoreMesh(core_axis_name="c", subcore_axis_name="s")

  scratch = (
      [pltpu.VMEM((CH, D), jnp.float32) for _ in range(NBUF)]
      + [pltpu.VMEM((CH,), jnp.int32) for _ in range(NBUF)]
      + [pltpu.VMEM((S, D), jnp.float32)]
      + [pltpu.SemaphoreType.DMA for _ in range(2 * NBUF)]
  )

  @functools.partial(
      pl.kernel,
      mesh=mesh,
      out_type=jax.ShapeDtypeStruct((N, D), jnp.float32),
      scratch_types=scratch,
      compiler_params=pltpu.CompilerParams(use_tc_tiling_on_sc=False),
  )
  def emb(table, idx_hbm, pos_hbm, out_hbm, *scr):
    rows = scr[0:NBUF]
    idxb = scr[NBUF:2 * NBUF]
    pos_v = scr[2 * NBUF]
    gsem = scr[2 * NBUF + 1: 2 * NBUF + 1 + NBUF]
    osem = scr[2 * NBUF + 1 + NBUF: 2 * NBUF + 1 + 2 * NBUF]

    wid = lax.axis_index("s") * NC + lax.axis_index("c")
    base = wid * NR

    pltpu.sync_copy(pos_hbm, pos_v)

    def fire_gathers(g, b):
      r0 = base + g * CH
      pltpu.sync_copy(idx_hbm.at[pl.ds(r0, CH)], idxb[b])
      for kk in range(NG):
        pltpu.async_copy(
            table.at[idxb[b].at[pl.ds(kk * K, K)]],
            rows[b].at[pl.ds(kk * K, K)],
            gsem[b],
        )

    def wait_gathers(b):
      for kk in range(NG):
        pltpu.make_async_copy(
            table.at[idxb[b].at[pl.ds(kk * K, K)]],
            rows[b].at[pl.ds(kk * K, K)],
            gsem[b],
        ).wait()

    def wait_outwrite(g, b):
      pltpu.make_async_copy(
          rows[b], out_hbm.at[pl.ds(base + g * CH, CH)], osem[b]
      ).wait()

    for b in range(NBUF - 1):
      fire_gathers(jnp.int32(b), b)

    def outer(i, carry):
      for b in range(NBUF):
        g = i * NBUF + b
        wait_gathers(b)

        def add_pos(j, c2, _rows=rows[b]):
          for c in range(D // NL):
            pv = pos_v[j, pl.ds(c * NL, NL)]
            for rep in range(REP):
              r = rep * S + j
              _rows[r, pl.ds(c * NL, NL)] = _rows[r, pl.ds(c * NL, NL)] + pv
          return c2
        lax.fori_loop(0, S, add_pos, 0)

        pltpu.async_copy(
            rows[b], out_hbm.at[pl.ds(base + g * CH, CH)], osem[b]
        )

        gp = g + (NBUF - 1)
        bp = (b + NBUF - 1) % NBUF

        @pl.when(jnp.logical_and(gp < NCH, g >= 1))
        def _():
          wait_outwrite(g - 1, bp)

        @pl.when(gp < NCH)
        def _():
          fire_gathers(gp, bp)
      return carry

    lax.fori_loop(0, NCH // NBUF, outer, 0)

    for b in range(NBUF):
      wait_outwrite(NCH - NBUF + b, b)

  return emb


def kernel(input_ids, token_table, position_table):
  B, S = input_ids.shape
  V, D = token_table.shape
  N = B * S
  info = plsc.get_sparse_core_info()
  emb = _build(V, D, N, S, info.num_cores, info.num_subcores)
  idx = jnp.reshape(input_ids, (N,)).astype(jnp.int32)
  out = emb(token_table, idx, position_table)
  return jnp.reshape(out, (B, S, D))
